# baseline (speedup 1.0000x reference)
import jax
import jax.numpy as jnp
from jax import lax
from jax.experimental import pallas as pl
from jax.experimental.pallas import tpu as pltpu
from jax.experimental.pallas import tpu_sc as plsc

_NC = 2
_NS = 16
_NW = _NC * _NS
_CH_ROWS = 32


def _add_kernel(x_ref, t_ref, o_ref):
    o_ref[...] = x_ref[...] + t_ref[...][None, :, :]


def _tc_add(inputs, table):
    B, S, D = inputs.shape
    S_BLK = 512
    return pl.pallas_call(
        _add_kernel,
        grid=(S // S_BLK,),
        in_specs=[
            pl.BlockSpec((B, S_BLK, D), lambda i: (0, i, 0)),
            pl.BlockSpec((S_BLK, D), lambda i: (i, 0)),
        ],
        out_specs=pl.BlockSpec((B, S_BLK, D), lambda i: (0, i, 0)),
        out_shape=jax.ShapeDtypeStruct((B, S, D), inputs.dtype),
    )(inputs, table)


def _sc_add(in_hbm, tbl_hbm, out_hbm, tbl_v, in_v0, in_v1,
            ld0, ld1, st0, st1, tb):
    rows_per_w = 8192 // _NW
    flat_per_w = rows_per_w * 1024
    ch = _CH_ROWS * 1024
    n_ch = rows_per_w // _CH_ROWS
    n_items = n_ch

    bufs = (in_v0, in_v1)
    lds = (ld0, ld1)
    sts = (st0, st1)

    wid = lax.axis_index("s") * _NC + lax.axis_index("c")
    base = wid * flat_per_w

    def chunk_off(c):
        return base + c * ch

    tbl_dma = pltpu.async_copy(tbl_hbm.at[pl.ds(chunk_off(0), ch)], tbl_v, tb)
    ld_dma = {0: pltpu.async_copy(in_hbm.at[3, pl.ds(chunk_off(0), ch)],
                                  bufs[0], lds[0])}
    st_dma = {}

    for k in range(n_items):
        kb = k % 2
        tbl_dma.wait()
        ld_dma.pop(k).wait()
        buf = bufs[kb]

        @plsc.parallel_loop(0, ch, step=16, unroll=8)
        def _add(i):
            plsc.addupdate(buf.at[pl.ds(i, 16)], tbl_v[pl.ds(i, 16)])

        if k + 1 < n_items:
            tbl_dma = pltpu.async_copy(
                tbl_hbm.at[pl.ds(chunk_off(k + 1), ch)], tbl_v, tb)

        st_dma[k] = pltpu.async_copy(
            buf, out_hbm.at[0, pl.ds(chunk_off(k), ch)], sts[kb])

        if k + 1 < n_items:
            if k - 1 >= 0:
                st_dma.pop(k - 1).wait()
            ld_dma[k + 1] = pltpu.async_copy(
                in_hbm.at[3, pl.ds(chunk_off(k + 1), ch)],
                bufs[(k + 1) % 2], lds[(k + 1) % 2])

    st_dma.pop(n_items - 1).wait()


def kernel(inputs, table):
    B, S, D = inputs.shape
    tc_out = _tc_add(inputs, table)
    flat_in = inputs.reshape(B, S * D)
    flat_tbl = table.reshape(S * D)
    sc_call = pl.kernel(
        _sc_add,
        out_type=jax.ShapeDtypeStruct((1, S * D), inputs.dtype),
        mesh=plsc.VectorSubcoreMesh(core_axis_name="c", subcore_axis_name="s"),
        scratch_types=[
            pltpu.VMEM((_CH_ROWS * 1024,), jnp.float32),
            pltpu.VMEM((_CH_ROWS * 1024,), jnp.float32),
            pltpu.VMEM((_CH_ROWS * 1024,), jnp.float32),
            pltpu.SemaphoreType.DMA,
            pltpu.SemaphoreType.DMA,
            pltpu.SemaphoreType.DMA,
            pltpu.SemaphoreType.DMA,
            pltpu.SemaphoreType.DMA,
        ],
    )
    sc_out = sc_call(flat_in, flat_tbl)
    return tc_out, sc_out


# speedup vs baseline: 2.8636x; 2.8636x over previous
import jax
import jax.numpy as jnp
from jax.experimental import pallas as pl


def _add_kernel(x_ref, t_ref, o_ref):
    o_ref[...] = x_ref[...] + t_ref[...][None, :, :]


def kernel(inputs, table):
    B, S, D = inputs.shape
    S_BLK = 512
    grid = (S // S_BLK,)
    return pl.pallas_call(
        _add_kernel,
        grid=grid,
        in_specs=[
            pl.BlockSpec((B, S_BLK, D), lambda i: (0, i, 0)),
            pl.BlockSpec((S_BLK, D), lambda i: (i, 0)),
        ],
        out_specs=pl.BlockSpec((B, S_BLK, D), lambda i: (0, i, 0)),
        out_shape=jax.ShapeDtypeStruct((B, S, D), inputs.dtype),
    )(inputs, table)
